# Initial kernel scaffold; baseline (speedup 1.0000x reference)
import jax
import jax.numpy as jnp
from jax.experimental import pallas as pl


def kernel(x, edge_index, W1, b1, W2, b2):
    raise NotImplementedError("write your pallas kernel here")


# speedup vs baseline: 1.7211x; 1.7211x over previous
import functools
import jax
import jax.numpy as jnp
from jax.experimental import pallas as pl

K = 3
N = 10000
E = 320000
D_IN = 128
HID = 16
D_OUT = 128


def _final_stage(acc2_ref, g2w_ref, o_ref):
    z = acc2_ref[...] + 2.0 * g2w_ref[...]
    m = jnp.max(z, axis=1, keepdims=True)
    ez = jnp.exp(z - m)
    lse = jnp.log(jnp.sum(ez, axis=1, keepdims=True))
    o_ref[...] = z - m - lse


def kernel(x, edge_index, W1, b1, W2, b2):
    row = edge_index[0]
    col = edge_index[1]

    deg = jnp.zeros((N,), jnp.float32).at[row].add(1.0)
    dis = jnp.where(deg > 0, 1.0 / jnp.sqrt(jnp.where(deg > 0, deg, 1.0)), 0.0)
    d = dis[:, None]

    def S(v):
        return jnp.zeros((N, v.shape[1]), v.dtype).at[col].add(v[row])

    p2 = d * (x @ W1[2])
    q2 = S(p2)
    u1 = x @ W1[1]
    base1 = x @ (W1[0] - W1[2]) + b1
    p1 = d * u1 - 2.0 * (d * d) * q2
    s = S(p1)
    h = jax.nn.relu(base1 - d * s)
    ph = d * h
    qh = S(ph)
    g1 = -d * qh
    base2 = h @ (W2[0] - W2[2]) + b2
    acc2 = base2 + g1 @ W2[1]
    pg = (d * d) * qh
    r = S(pg)
    g2 = d * r
    g2w = g2 @ W2[2]

    B = 2000
    out = pl.pallas_call(
        _final_stage,
        grid=(N // B,),
        in_specs=[
            pl.BlockSpec((B, D_OUT), lambda i: (i, 0)),
            pl.BlockSpec((B, D_OUT), lambda i: (i, 0)),
        ],
        out_specs=pl.BlockSpec((B, D_OUT), lambda i: (i, 0)),
        out_shape=jax.ShapeDtypeStruct((N, D_OUT), jnp.float32),
    )(acc2, g2w)
    return out


# speedup vs baseline: 9.6202x; 5.5896x over previous
import functools
import jax
import jax.numpy as jnp
from jax import lax
from jax.experimental import pallas as pl
from jax.experimental.pallas import tpu as pltpu
from jax.experimental.pallas import tpu_sc as plsc

K = 3
N = 10000
E = 320000
D_IN = 128
HID = 16
D_OUT = 128

NC = 2
NS = 16
NW = NC * NS
EW = E // NW
C = 80
NCH = EW // C
NP = 10240
ROWS_PER_TILE = NP // NS

_mesh = plsc.VectorSubcoreMesh(core_axis_name="c", subcore_axis_name="s")


def _zero_fill(zbuf):
    def body(j, carry):
        zbuf[j, :] = jnp.zeros((16,), jnp.float32)
        return carry
    lax.fori_loop(0, zbuf.shape[0], body, 0)


@functools.partial(
    pl.kernel,
    mesh=_mesh,
    out_type=jax.ShapeDtypeStruct((NC, NP, HID), jnp.float32),
    scratch_types=[
        pltpu.VMEM((C,), jnp.int32),
        pltpu.VMEM((C,), jnp.int32),
        pltpu.VMEM((C, HID), jnp.float32),
        pltpu.VMEM((ROWS_PER_TILE, HID), jnp.float32),
        pltpu.VMEM_SHARED((NP, HID), jnp.float32),
        pltpu.SemaphoreType.DMA,
    ],
    compiler_params=pltpu.CompilerParams(use_tc_tiling_on_sc=False),
)
def _scatter_pass(v_hbm, src_hbm, dst_hbm, out_hbm,
                  sidx, didx, buf, zbuf, acc, sem):
    c = lax.axis_index("c")
    s = lax.axis_index("s")
    w = c * NS + s

    _zero_fill(zbuf)
    pltpu.sync_copy(zbuf, acc.at[pl.ds(s * ROWS_PER_TILE, ROWS_PER_TILE)])
    plsc.subcore_barrier()

    def body(i, carry):
        base = w * EW + i * C
        pltpu.sync_copy(src_hbm.at[pl.ds(base, C)], sidx)
        pltpu.sync_copy(dst_hbm.at[pl.ds(base, C)], didx)
        pltpu.async_copy(v_hbm.at[sidx], buf, sem).wait()
        pltpu.sync_copy(buf, acc.at[didx], add=True)
        return carry

    lax.fori_loop(0, NCH, body, 0)
    plsc.subcore_barrier()

    pltpu.sync_copy(acc.at[pl.ds(s * ROWS_PER_TILE, ROWS_PER_TILE)],
                    out_hbm.at[c, pl.ds(s * ROWS_PER_TILE, ROWS_PER_TILE)])


@functools.partial(
    pl.kernel,
    mesh=_mesh,
    out_type=jax.ShapeDtypeStruct((NC, NP, HID), jnp.float32),
    scratch_types=[
        pltpu.VMEM((C,), jnp.int32),
        pltpu.VMEM((C, HID), jnp.float32),
        pltpu.VMEM((ROWS_PER_TILE, HID), jnp.float32),
        pltpu.VMEM_SHARED((NP, HID), jnp.float32),
    ],
    compiler_params=pltpu.CompilerParams(use_tc_tiling_on_sc=False),
)
def _degree_pass(dst_hbm, out_hbm, didx, ones_buf, zbuf, acc):
    c = lax.axis_index("c")
    s = lax.axis_index("s")
    w = c * NS + s

    _zero_fill(zbuf)
    pltpu.sync_copy(zbuf, acc.at[pl.ds(s * ROWS_PER_TILE, ROWS_PER_TILE)])

    def fill(j, carry):
        ones_buf[j, :] = jnp.ones((16,), jnp.float32)
        return carry
    lax.fori_loop(0, C, fill, 0)
    plsc.subcore_barrier()

    def body(i, carry):
        base = w * EW + i * C
        pltpu.sync_copy(dst_hbm.at[pl.ds(base, C)], didx)
        pltpu.sync_copy(ones_buf, acc.at[didx], add=True)
        return carry

    lax.fori_loop(0, NCH, body, 0)
    plsc.subcore_barrier()

    pltpu.sync_copy(acc.at[pl.ds(s * ROWS_PER_TILE, ROWS_PER_TILE)],
                    out_hbm.at[c, pl.ds(s * ROWS_PER_TILE, ROWS_PER_TILE)])


TB = 2000


def _tc_call(fn, out_shapes, in_specs, out_specs, *args):
    return pl.pallas_call(
        fn,
        grid=(N // TB,),
        in_specs=in_specs,
        out_specs=out_specs,
        out_shape=out_shapes,
    )(*args)


def _bs(r, c_):
    return pl.BlockSpec((TB, c_), lambda i: (i, 0)) if r else None


_full = lambda shape: pl.BlockSpec(shape, lambda i: tuple(0 for _ in shape))
_rows = lambda c_: pl.BlockSpec((TB, c_), lambda i: (i, 0))
_prows = lambda c_: pl.BlockSpec((NC, TB, c_), lambda i: (0, i, 0))


def _stage1(degp_ref, x_ref, w1_ref, b1_ref, disb_ref, p2_ref, u1_ref, base1_ref):
    deg = degp_ref[0, :, 0:1] + degp_ref[1, :, 0:1]
    dis = jnp.where(deg > 0, lax.rsqrt(jnp.where(deg > 0, deg, 1.0)), 0.0)
    disb = jnp.broadcast_to(dis, (TB, HID))
    disb_ref[...] = disb
    x = x_ref[...]
    w1 = w1_ref[...]
    p2_ref[...] = disb * jnp.dot(x, w1[2], preferred_element_type=jnp.float32)
    u1_ref[...] = jnp.dot(x, w1[1], preferred_element_type=jnp.float32)
    base1_ref[...] = (
        jnp.dot(x, w1[0] - w1[2], preferred_element_type=jnp.float32)
        + b1_ref[0:1, :]
    )


def _stage2(u1_ref, q2p_ref, disb_ref, p1_ref):
    disb = disb_ref[...]
    q2 = q2p_ref[0] + q2p_ref[1]
    p1_ref[...] = disb * u1_ref[...] - 2.0 * disb * disb * q2


def _stage3(base1_ref, sp_ref, disb_ref, w2_ref, b2_ref, ph_ref, base2_ref):
    disb = disb_ref[...]
    s = sp_ref[0] + sp_ref[1]
    h = jnp.maximum(base1_ref[...] - disb * s, 0.0)
    ph_ref[...] = disb * h
    w2 = w2_ref[...]
    base2_ref[...] = (
        jnp.dot(h, w2[0] - w2[2], preferred_element_type=jnp.float32)
        + b2_ref[0:1, :]
    )


def _stage4(qhp_ref, disb_ref, base2_ref, w2_ref, acc2_ref, pg_ref):
    disb = disb_ref[...]
    qh = qhp_ref[0] + qhp_ref[1]
    g1 = -disb * qh
    acc2_ref[...] = base2_ref[...] + jnp.dot(
        g1, w2_ref[...][1], preferred_element_type=jnp.float32)
    pg_ref[...] = disb * disb * qh


def _stage5(rp_ref, disb_ref, acc2_ref, w2_ref, o_ref):
    disb = disb_ref[...]
    g2 = disb * (rp_ref[0] + rp_ref[1])
    z = acc2_ref[...] + 2.0 * jnp.dot(
        g2, w2_ref[...][2], preferred_element_type=jnp.float32)
    m = jnp.max(z, axis=1, keepdims=True)
    ez = jnp.exp(z - m)
    lse = jnp.log(jnp.sum(ez, axis=1, keepdims=True))
    o_ref[...] = z - m - lse


def kernel(x, edge_index, W1, b1, W2, b2):
    row = edge_index[0]
    col = edge_index[1]
    b1r = jnp.broadcast_to(b1[None, :], (8, HID))
    b2r = jnp.broadcast_to(b2[None, :], (8, D_OUT))

    degp = _degree_pass(row)

    f32 = jnp.float32
    disb, p2, u1, base1 = _tc_call(
        _stage1,
        [jax.ShapeDtypeStruct((N, HID), f32)] * 4,
        [_prows(HID), _rows(D_IN), _full((K, D_IN, HID)), _full((8, HID))],
        [_rows(HID)] * 4,
        degp, x, W1, b1r)

    q2p = _scatter_pass(p2, row, col)

    p1 = _tc_call(
        _stage2,
        jax.ShapeDtypeStruct((N, HID), f32),
        [_rows(HID), _prows(HID), _rows(HID)],
        _rows(HID),
        u1, q2p, disb)

    sp = _scatter_pass(p1, row, col)

    ph, base2 = _tc_call(
        _stage3,
        [jax.ShapeDtypeStruct((N, HID), f32),
         jax.ShapeDtypeStruct((N, D_OUT), f32)],
        [_rows(HID), _prows(HID), _rows(HID), _full((K, HID, D_OUT)),
         _full((8, D_OUT))],
        [_rows(HID), _rows(D_OUT)],
        base1, sp, disb, W2, b2r)

    qhp = _scatter_pass(ph, row, col)

    acc2, pg = _tc_call(
        _stage4,
        [jax.ShapeDtypeStruct((N, D_OUT), f32),
         jax.ShapeDtypeStruct((N, HID), f32)],
        [_prows(HID), _rows(HID), _rows(D_OUT), _full((K, HID, D_OUT))],
        [_rows(D_OUT), _rows(HID)],
        qhp, disb, base2, W2)

    rp = _scatter_pass(pg, row, col)

    out = _tc_call(
        _stage5,
        jax.ShapeDtypeStruct((N, D_OUT), f32),
        [_prows(HID), _rows(HID), _rows(D_OUT), _full((K, HID, D_OUT))],
        _rows(D_OUT),
        rp, disb, acc2, W2)

    return out


# speedup vs baseline: 18.8210x; 1.9564x over previous
import functools
import jax
import jax.numpy as jnp
from jax import lax
from jax.experimental import pallas as pl
from jax.experimental.pallas import tpu as pltpu
from jax.experimental.pallas import tpu_sc as plsc

K = 3
N = 10000
E = 320000
D_IN = 128
HID = 16
D_OUT = 128

NC = 2
NS = 16
NW = NC * NS
EW = E // NW
C = 40
NCH = EW // C
NP = 10240
RPT = NP // NS

_mesh = plsc.VectorSubcoreMesh(core_axis_name="c", subcore_axis_name="s")

_SC_SCRATCH = [
    pltpu.VMEM((NCH, C), jnp.int32),
    pltpu.VMEM((NCH, C), jnp.int32),
    pltpu.VMEM((C, HID), jnp.float32),
    pltpu.VMEM((C, HID), jnp.float32),
    pltpu.VMEM_SHARED((NP, HID), jnp.float32),
    pltpu.SemaphoreType.DMA,
    pltpu.SemaphoreType.DMA,
    pltpu.SemaphoreType.DMA,
    pltpu.SemaphoreType.DMA,
]


@functools.partial(
    pl.kernel,
    mesh=_mesh,
    out_type=jax.ShapeDtypeStruct((NC, NP, HID), jnp.float32),
    scratch_types=_SC_SCRATCH,
    compiler_params=pltpu.CompilerParams(use_tc_tiling_on_sc=False),
)
def _scatter_pass(v_hbm, src3_hbm, dst3_hbm, zeros_hbm, out_hbm,
                  sidx_all, didx_all, buf0, buf1, acc,
                  gsem0, gsem1, ssem0, ssem1):
    c = lax.axis_index("c")
    s = lax.axis_index("s")
    w = c * NS + s

    pltpu.sync_copy(zeros_hbm.at[pl.ds(s * RPT, RPT)],
                    acc.at[pl.ds(s * RPT, RPT)])
    pltpu.sync_copy(src3_hbm.at[w], sidx_all)
    pltpu.sync_copy(dst3_hbm.at[w], didx_all)
    plsc.subcore_barrier()

    bufs = (buf0, buf1)
    gsems = (gsem0, gsem1)
    ssems = (ssem0, ssem1)

    def fire_gather(i, p):
        pltpu.async_copy(v_hbm.at[sidx_all.at[i]], bufs[p], gsems[p])

    def drain_gather(p):
        pltpu.make_async_copy(v_hbm.at[pl.ds(0, C)], bufs[p], gsems[p]).wait()

    def fire_scatter(i, p):
        pltpu.async_copy(bufs[p], acc.at[didx_all.at[i]], ssems[p], add=True)

    def drain_scatter(p):
        pltpu.make_async_copy(v_hbm.at[pl.ds(0, C)], bufs[p], ssems[p]).wait()

    fire_gather(0, 0)

    def body(k, carry):
        a = 2 * k
        b = a + 1

        @pl.when(k > 0)
        def _():
            drain_scatter(1)
        fire_gather(b, 1)
        drain_gather(0)
        fire_scatter(a, 0)

        drain_scatter(0)

        @pl.when(b + 1 < NCH)
        def _():
            fire_gather(b + 1, 0)
        drain_gather(1)
        fire_scatter(b, 1)
        return carry

    lax.fori_loop(0, NCH // 2, body, 0)
    drain_scatter(1)
    plsc.subcore_barrier()

    pltpu.sync_copy(acc.at[pl.ds(s * RPT, RPT)],
                    out_hbm.at[c, pl.ds(s * RPT, RPT)])


@functools.partial(
    pl.kernel,
    mesh=_mesh,
    out_type=jax.ShapeDtypeStruct((NC, NP, HID), jnp.float32),
    scratch_types=[
        pltpu.VMEM((NCH, C), jnp.int32),
        pltpu.VMEM((C, HID), jnp.float32),
        pltpu.VMEM_SHARED((NP, HID), jnp.float32),
        pltpu.SemaphoreType.DMA,
        pltpu.SemaphoreType.DMA,
    ],
    compiler_params=pltpu.CompilerParams(use_tc_tiling_on_sc=False),
)
def _degree_pass(dst3_hbm, zeros_hbm, out_hbm,
                 didx_all, ones_buf, acc, ssem0, ssem1):
    c = lax.axis_index("c")
    s = lax.axis_index("s")
    w = c * NS + s

    pltpu.sync_copy(zeros_hbm.at[pl.ds(s * RPT, RPT)],
                    acc.at[pl.ds(s * RPT, RPT)])
    pltpu.sync_copy(dst3_hbm.at[w], didx_all)

    def fill(j, carry):
        ones_buf[j, :] = jnp.ones((16,), jnp.float32)
        return carry
    lax.fori_loop(0, C, fill, 0)
    plsc.subcore_barrier()

    ssems = (ssem0, ssem1)

    def fire_scatter(i, p):
        pltpu.async_copy(ones_buf, acc.at[didx_all.at[i]], ssems[p], add=True)

    def drain_scatter(p):
        pltpu.make_async_copy(zeros_hbm.at[pl.ds(0, C)], ones_buf,
                              ssems[p]).wait()

    def body(k, carry):
        a = 2 * k
        b = a + 1

        @pl.when(k > 0)
        def _():
            drain_scatter(0)
            drain_scatter(1)
        fire_scatter(a, 0)
        fire_scatter(b, 1)
        return carry

    lax.fori_loop(0, NCH // 2, body, 0)
    drain_scatter(0)
    drain_scatter(1)
    plsc.subcore_barrier()

    pltpu.sync_copy(acc.at[pl.ds(s * RPT, RPT)],
                    out_hbm.at[c, pl.ds(s * RPT, RPT)])


TB = 2000


def _tc_call(fn, out_shapes, in_specs, out_specs, *args):
    return pl.pallas_call(
        fn,
        grid=(N // TB,),
        in_specs=in_specs,
        out_specs=out_specs,
        out_shape=out_shapes,
    )(*args)


_full = lambda shape: pl.BlockSpec(shape, lambda i: tuple(0 for _ in shape))
_rows = lambda c_: pl.BlockSpec((TB, c_), lambda i: (i, 0))
_prows = lambda c_: pl.BlockSpec((NC, TB, c_), lambda i: (0, i, 0))


def _stage1(degp_ref, x_ref, w1_ref, b1_ref, disb_ref, p2_ref, u1_ref, base1_ref):
    deg = degp_ref[0, :, 0:1] + degp_ref[1, :, 0:1]
    dis = jnp.where(deg > 0, lax.rsqrt(jnp.where(deg > 0, deg, 1.0)), 0.0)
    disb = jnp.broadcast_to(dis, (TB, HID))
    disb_ref[...] = disb
    x = x_ref[...]
    w1 = w1_ref[...]
    p2_ref[...] = disb * jnp.dot(x, w1[2], preferred_element_type=jnp.float32)
    u1_ref[...] = jnp.dot(x, w1[1], preferred_element_type=jnp.float32)
    base1_ref[...] = (
        jnp.dot(x, w1[0] - w1[2], preferred_element_type=jnp.float32)
        + b1_ref[0:1, :]
    )


def _stage2(u1_ref, q2p_ref, disb_ref, p1_ref):
    disb = disb_ref[...]
    q2 = q2p_ref[0] + q2p_ref[1]
    p1_ref[...] = disb * u1_ref[...] - 2.0 * disb * disb * q2


def _stage3(base1_ref, sp_ref, disb_ref, w2_ref, b2_ref, ph_ref, base2_ref):
    disb = disb_ref[...]
    s = sp_ref[0] + sp_ref[1]
    h = jnp.maximum(base1_ref[...] - disb * s, 0.0)
    ph_ref[...] = disb * h
    w2 = w2_ref[...]
    base2_ref[...] = (
        jnp.dot(h, w2[0] - w2[2], preferred_element_type=jnp.float32)
        + b2_ref[0:1, :]
    )


def _stage4(qhp_ref, disb_ref, base2_ref, w2_ref, acc2_ref, pg_ref):
    disb = disb_ref[...]
    qh = qhp_ref[0] + qhp_ref[1]
    g1 = -disb * qh
    acc2_ref[...] = base2_ref[...] + jnp.dot(
        g1, w2_ref[...][1], preferred_element_type=jnp.float32)
    pg_ref[...] = disb * disb * qh


def _stage5(rp_ref, disb_ref, acc2_ref, w2_ref, o_ref):
    disb = disb_ref[...]
    g2 = disb * (rp_ref[0] + rp_ref[1])
    z = acc2_ref[...] + 2.0 * jnp.dot(
        g2, w2_ref[...][2], preferred_element_type=jnp.float32)
    m = jnp.max(z, axis=1, keepdims=True)
    ez = jnp.exp(z - m)
    lse = jnp.log(jnp.sum(ez, axis=1, keepdims=True))
    o_ref[...] = z - m - lse


def kernel(x, edge_index, W1, b1, W2, b2):
    row3 = edge_index[0].reshape(NW, NCH, C)
    col3 = edge_index[1].reshape(NW, NCH, C)
    zeros = jnp.zeros((NP, HID), jnp.float32)
    b1r = jnp.broadcast_to(b1[None, :], (8, HID))
    b2r = jnp.broadcast_to(b2[None, :], (8, D_OUT))

    degp = _degree_pass(row3, zeros)

    f32 = jnp.float32
    disb, p2, u1, base1 = _tc_call(
        _stage1,
        [jax.ShapeDtypeStruct((N, HID), f32)] * 4,
        [_prows(HID), _rows(D_IN), _full((K, D_IN, HID)), _full((8, HID))],
        [_rows(HID)] * 4,
        degp, x, W1, b1r)

    q2p = _scatter_pass(p2, row3, col3, zeros)

    p1 = _tc_call(
        _stage2,
        jax.ShapeDtypeStruct((N, HID), f32),
        [_rows(HID), _prows(HID), _rows(HID)],
        _rows(HID),
        u1, q2p, disb)

    sp = _scatter_pass(p1, row3, col3, zeros)

    ph, base2 = _tc_call(
        _stage3,
        [jax.ShapeDtypeStruct((N, HID), f32),
         jax.ShapeDtypeStruct((N, D_OUT), f32)],
        [_rows(HID), _prows(HID), _rows(HID), _full((K, HID, D_OUT)),
         _full((8, D_OUT))],
        [_rows(HID), _rows(D_OUT)],
        base1, sp, disb, W2, b2r)

    qhp = _scatter_pass(ph, row3, col3, zeros)

    acc2, pg = _tc_call(
        _stage4,
        [jax.ShapeDtypeStruct((N, D_OUT), f32),
         jax.ShapeDtypeStruct((N, HID), f32)],
        [_prows(HID), _rows(HID), _rows(D_OUT), _full((K, HID, D_OUT))],
        [_rows(D_OUT), _rows(HID)],
        qhp, disb, base2, W2)

    rp = _scatter_pass(pg, row3, col3, zeros)

    out = _tc_call(
        _stage5,
        jax.ShapeDtypeStruct((N, D_OUT), f32),
        [_prows(HID), _rows(HID), _rows(D_OUT), _full((K, HID, D_OUT))],
        _rows(D_OUT),
        rp, disb, acc2, W2)

    return out


# speedup vs baseline: 32.7107x; 1.7380x over previous
import functools
import jax
import jax.numpy as jnp
from jax import lax
from jax.experimental import pallas as pl
from jax.experimental.pallas import tpu as pltpu
from jax.experimental.pallas import tpu_sc as plsc

K = 3
N = 10000
E = 320000
D_IN = 128
HID = 16
D_OUT = 128

NC = 2
NS = 16
NW = NC * NS
EW = E // NW
C = 40
NCH = EW // C
NP = 10240
RPT = NP // NS

_mesh = plsc.VectorSubcoreMesh(core_axis_name="c", subcore_axis_name="s")

_SC_SCRATCH = [
    pltpu.VMEM((NCH, C), jnp.int32),
    pltpu.VMEM((NCH, C), jnp.int32),
    pltpu.VMEM((C, HID), jnp.float32),
    pltpu.VMEM((C, HID), jnp.float32),
    pltpu.VMEM_SHARED((NP, HID), jnp.float32),
    pltpu.VMEM_SHARED((N, HID), jnp.float32),
    pltpu.SemaphoreType.DMA,
    pltpu.SemaphoreType.DMA,
    pltpu.SemaphoreType.DMA,
    pltpu.SemaphoreType.DMA,
]
VROWS = N // NS


@functools.partial(
    pl.kernel,
    mesh=_mesh,
    out_type=jax.ShapeDtypeStruct((NC, NP, HID), jnp.float32),
    scratch_types=_SC_SCRATCH,
    compiler_params=pltpu.CompilerParams(use_tc_tiling_on_sc=False),
)
def _scatter_pass(v_hbm, src3_hbm, dst3_hbm, zeros_hbm, out_hbm,
                  sidx_all, didx_all, buf0, buf1, acc, vspm,
                  gsem0, gsem1, ssem0, ssem1):
    c = lax.axis_index("c")
    s = lax.axis_index("s")
    w = c * NS + s

    pltpu.sync_copy(zeros_hbm.at[pl.ds(s * RPT, RPT)],
                    acc.at[pl.ds(s * RPT, RPT)])
    pltpu.sync_copy(v_hbm.at[pl.ds(s * VROWS, VROWS)],
                    vspm.at[pl.ds(s * VROWS, VROWS)])
    pltpu.sync_copy(src3_hbm.at[w], sidx_all)
    pltpu.sync_copy(dst3_hbm.at[w], didx_all)
    plsc.subcore_barrier()

    bufs = (buf0, buf1)
    gsems = (gsem0, gsem1)
    ssems = (ssem0, ssem1)

    def fire_gather(i, p):
        pltpu.async_copy(vspm.at[sidx_all.at[i]], bufs[p], gsems[p])

    def drain_gather(p):
        pltpu.make_async_copy(v_hbm.at[pl.ds(0, C)], bufs[p], gsems[p]).wait()

    def fire_scatter(i, p):
        pltpu.async_copy(bufs[p], acc.at[didx_all.at[i]], ssems[p], add=True)

    def drain_scatter(p):
        pltpu.make_async_copy(v_hbm.at[pl.ds(0, C)], bufs[p], ssems[p]).wait()

    fire_gather(0, 0)

    def body(k, carry):
        a = 2 * k
        b = a + 1

        @pl.when(k > 0)
        def _():
            drain_scatter(1)
        fire_gather(b, 1)
        drain_gather(0)
        fire_scatter(a, 0)

        drain_scatter(0)

        @pl.when(b + 1 < NCH)
        def _():
            fire_gather(b + 1, 0)
        drain_gather(1)
        fire_scatter(b, 1)
        return carry

    lax.fori_loop(0, NCH // 2, body, 0)
    drain_scatter(1)
    plsc.subcore_barrier()

    pltpu.sync_copy(acc.at[pl.ds(s * RPT, RPT)],
                    out_hbm.at[c, pl.ds(s * RPT, RPT)])


@functools.partial(
    pl.kernel,
    mesh=_mesh,
    out_type=jax.ShapeDtypeStruct((NC, NP, HID), jnp.float32),
    scratch_types=[
        pltpu.VMEM((NCH, C), jnp.int32),
        pltpu.VMEM((C, HID), jnp.float32),
        pltpu.VMEM_SHARED((NP, HID), jnp.float32),
        pltpu.SemaphoreType.DMA,
        pltpu.SemaphoreType.DMA,
    ],
    compiler_params=pltpu.CompilerParams(use_tc_tiling_on_sc=False),
)
def _degree_pass(dst3_hbm, zeros_hbm, out_hbm,
                 didx_all, ones_buf, acc, ssem0, ssem1):
    c = lax.axis_index("c")
    s = lax.axis_index("s")
    w = c * NS + s

    pltpu.sync_copy(zeros_hbm.at[pl.ds(s * RPT, RPT)],
                    acc.at[pl.ds(s * RPT, RPT)])
    pltpu.sync_copy(dst3_hbm.at[w], didx_all)

    def fill(j, carry):
        ones_buf[j, :] = jnp.ones((16,), jnp.float32)
        return carry
    lax.fori_loop(0, C, fill, 0)
    plsc.subcore_barrier()

    ssems = (ssem0, ssem1)

    def fire_scatter(i, p):
        pltpu.async_copy(ones_buf, acc.at[didx_all.at[i]], ssems[p], add=True)

    def drain_scatter(p):
        pltpu.make_async_copy(zeros_hbm.at[pl.ds(0, C)], ones_buf,
                              ssems[p]).wait()

    def body(k, carry):
        a = 2 * k
        b = a + 1

        @pl.when(k > 0)
        def _():
            drain_scatter(0)
            drain_scatter(1)
        fire_scatter(a, 0)
        fire_scatter(b, 1)
        return carry

    lax.fori_loop(0, NCH // 2, body, 0)
    drain_scatter(0)
    drain_scatter(1)
    plsc.subcore_barrier()

    pltpu.sync_copy(acc.at[pl.ds(s * RPT, RPT)],
                    out_hbm.at[c, pl.ds(s * RPT, RPT)])


TB = 2000


def _tc_call(fn, out_shapes, in_specs, out_specs, *args):
    return pl.pallas_call(
        fn,
        grid=(N // TB,),
        in_specs=in_specs,
        out_specs=out_specs,
        out_shape=out_shapes,
    )(*args)


_full = lambda shape: pl.BlockSpec(shape, lambda i: tuple(0 for _ in shape))
_rows = lambda c_: pl.BlockSpec((TB, c_), lambda i: (i, 0))
_prows = lambda c_: pl.BlockSpec((NC, TB, c_), lambda i: (0, i, 0))


def _stage1(degp_ref, x_ref, w1_ref, b1_ref, disb_ref, p2_ref, u1_ref, base1_ref):
    deg = degp_ref[0, :, 0:1] + degp_ref[1, :, 0:1]
    dis = jnp.where(deg > 0, lax.rsqrt(jnp.where(deg > 0, deg, 1.0)), 0.0)
    disb = jnp.broadcast_to(dis, (TB, HID))
    disb_ref[...] = disb
    x = x_ref[...]
    w1 = w1_ref[...]
    p2_ref[...] = disb * jnp.dot(x, w1[2], preferred_element_type=jnp.float32)
    u1_ref[...] = jnp.dot(x, w1[1], preferred_element_type=jnp.float32)
    base1_ref[...] = (
        jnp.dot(x, w1[0] - w1[2], preferred_element_type=jnp.float32)
        + b1_ref[0:1, :]
    )


def _stage2(u1_ref, q2p_ref, disb_ref, p1_ref):
    disb = disb_ref[...]
    q2 = q2p_ref[0] + q2p_ref[1]
    p1_ref[...] = disb * u1_ref[...] - 2.0 * disb * disb * q2


def _stage3(base1_ref, sp_ref, disb_ref, w2_ref, b2_ref, ph_ref, base2_ref):
    disb = disb_ref[...]
    s = sp_ref[0] + sp_ref[1]
    h = jnp.maximum(base1_ref[...] - disb * s, 0.0)
    ph_ref[...] = disb * h
    w2 = w2_ref[...]
    base2_ref[...] = (
        jnp.dot(h, w2[0] - w2[2], preferred_element_type=jnp.float32)
        + b2_ref[0:1, :]
    )


def _stage4(qhp_ref, disb_ref, base2_ref, w2_ref, acc2_ref, pg_ref):
    disb = disb_ref[...]
    qh = qhp_ref[0] + qhp_ref[1]
    g1 = -disb * qh
    acc2_ref[...] = base2_ref[...] + jnp.dot(
        g1, w2_ref[...][1], preferred_element_type=jnp.float32)
    pg_ref[...] = disb * disb * qh


def _stage5(rp_ref, disb_ref, acc2_ref, w2_ref, o_ref):
    disb = disb_ref[...]
    g2 = disb * (rp_ref[0] + rp_ref[1])
    z = acc2_ref[...] + 2.0 * jnp.dot(
        g2, w2_ref[...][2], preferred_element_type=jnp.float32)
    m = jnp.max(z, axis=1, keepdims=True)
    ez = jnp.exp(z - m)
    lse = jnp.log(jnp.sum(ez, axis=1, keepdims=True))
    o_ref[...] = z - m - lse


def kernel(x, edge_index, W1, b1, W2, b2):
    row3 = edge_index[0].reshape(NW, NCH, C)
    col3 = edge_index[1].reshape(NW, NCH, C)
    zeros = jnp.zeros((NP, HID), jnp.float32)
    b1r = jnp.broadcast_to(b1[None, :], (8, HID))
    b2r = jnp.broadcast_to(b2[None, :], (8, D_OUT))

    degp = _degree_pass(row3, zeros)

    f32 = jnp.float32
    disb, p2, u1, base1 = _tc_call(
        _stage1,
        [jax.ShapeDtypeStruct((N, HID), f32)] * 4,
        [_prows(HID), _rows(D_IN), _full((K, D_IN, HID)), _full((8, HID))],
        [_rows(HID)] * 4,
        degp, x, W1, b1r)

    q2p = _scatter_pass(p2, row3, col3, zeros)

    p1 = _tc_call(
        _stage2,
        jax.ShapeDtypeStruct((N, HID), f32),
        [_rows(HID), _prows(HID), _rows(HID)],
        _rows(HID),
        u1, q2p, disb)

    sp = _scatter_pass(p1, row3, col3, zeros)

    ph, base2 = _tc_call(
        _stage3,
        [jax.ShapeDtypeStruct((N, HID), f32),
         jax.ShapeDtypeStruct((N, D_OUT), f32)],
        [_rows(HID), _prows(HID), _rows(HID), _full((K, HID, D_OUT)),
         _full((8, D_OUT))],
        [_rows(HID), _rows(D_OUT)],
        base1, sp, disb, W2, b2r)

    qhp = _scatter_pass(ph, row3, col3, zeros)

    acc2, pg = _tc_call(
        _stage4,
        [jax.ShapeDtypeStruct((N, D_OUT), f32),
         jax.ShapeDtypeStruct((N, HID), f32)],
        [_prows(HID), _rows(HID), _rows(D_OUT), _full((K, HID, D_OUT))],
        [_rows(D_OUT), _rows(HID)],
        qhp, disb, base2, W2)

    rp = _scatter_pass(pg, row3, col3, zeros)

    out = _tc_call(
        _stage5,
        jax.ShapeDtypeStruct((N, D_OUT), f32),
        [_prows(HID), _rows(HID), _rows(D_OUT), _full((K, HID, D_OUT))],
        _rows(D_OUT),
        rp, disb, acc2, W2)

    return out


# speedup vs baseline: 37.0122x; 1.1315x over previous
import functools
import jax
import jax.numpy as jnp
from jax import lax
from jax.experimental import pallas as pl
from jax.experimental.pallas import tpu as pltpu
from jax.experimental.pallas import tpu_sc as plsc

K = 3
N = 10000
E = 320000
D_IN = 128
HID = 16
D_OUT = 128

NC = 2
NS = 16
NW = NC * NS
EW = E // NW
C = 80
NCH = EW // C
NP = 10240
RPT = NP // NS

_mesh = plsc.VectorSubcoreMesh(core_axis_name="c", subcore_axis_name="s")

_SC_SCRATCH = [
    pltpu.VMEM((NCH, C), jnp.int32),
    pltpu.VMEM((NCH, C), jnp.int32),
    pltpu.VMEM((C, HID), jnp.float32),
    pltpu.VMEM((C, HID), jnp.float32),
    pltpu.VMEM_SHARED((NP, HID), jnp.float32),
    pltpu.VMEM_SHARED((N, HID), jnp.float32),
    pltpu.SemaphoreType.DMA,
    pltpu.SemaphoreType.DMA,
    pltpu.SemaphoreType.DMA,
    pltpu.SemaphoreType.DMA,
]
VROWS = N // NS


@functools.partial(
    pl.kernel,
    mesh=_mesh,
    out_type=jax.ShapeDtypeStruct((NC, NP, HID), jnp.float32),
    scratch_types=_SC_SCRATCH,
    compiler_params=pltpu.CompilerParams(use_tc_tiling_on_sc=False),
)
def _scatter_pass(v_hbm, src3_hbm, dst3_hbm, zeros_hbm, out_hbm,
                  sidx_all, didx_all, buf0, buf1, acc, vspm,
                  gsem0, gsem1, ssem0, ssem1):
    c = lax.axis_index("c")
    s = lax.axis_index("s")
    w = c * NS + s

    pltpu.sync_copy(zeros_hbm.at[pl.ds(s * RPT, RPT)],
                    acc.at[pl.ds(s * RPT, RPT)])
    pltpu.sync_copy(v_hbm.at[pl.ds(s * VROWS, VROWS)],
                    vspm.at[pl.ds(s * VROWS, VROWS)])
    pltpu.sync_copy(src3_hbm.at[w], sidx_all)
    pltpu.sync_copy(dst3_hbm.at[w], didx_all)
    plsc.subcore_barrier()

    bufs = (buf0, buf1)
    gsems = (gsem0, gsem1)
    ssems = (ssem0, ssem1)

    def fire_gather(i, p):
        pltpu.async_copy(vspm.at[sidx_all.at[i]], bufs[p], gsems[p])

    def drain_gather(p):
        pltpu.make_async_copy(v_hbm.at[pl.ds(0, C)], bufs[p], gsems[p]).wait()

    def fire_scatter(i, p):
        pltpu.async_copy(bufs[p], acc.at[didx_all.at[i]], ssems[p], add=True)

    def drain_scatter(p):
        pltpu.make_async_copy(v_hbm.at[pl.ds(0, C)], bufs[p], ssems[p]).wait()

    fire_gather(0, 0)
    fire_gather(1, 1)
    drain_gather(0)
    fire_scatter(0, 0)

    def body(k, carry):
        a = 2 * k + 1
        b = a + 1

        drain_scatter(0)
        fire_gather(a + 1, 0)
        drain_gather(1)
        fire_scatter(a, 1)

        drain_scatter(1)

        @pl.when(b + 1 < NCH)
        def _():
            fire_gather(b + 1, 1)
        drain_gather(0)
        fire_scatter(b, 0)
        return carry

    lax.fori_loop(0, (NCH - 1) // 2, body, 0)
    drain_scatter(0)
    plsc.subcore_barrier()

    pltpu.sync_copy(acc.at[pl.ds(s * RPT, RPT)],
                    out_hbm.at[c, pl.ds(s * RPT, RPT)])


@functools.partial(
    pl.kernel,
    mesh=_mesh,
    out_type=jax.ShapeDtypeStruct((NC, NP, HID), jnp.float32),
    scratch_types=[
        pltpu.VMEM((NCH, C), jnp.int32),
        pltpu.VMEM((C, HID), jnp.float32),
        pltpu.VMEM_SHARED((NP, HID), jnp.float32),
        pltpu.SemaphoreType.DMA,
        pltpu.SemaphoreType.DMA,
    ],
    compiler_params=pltpu.CompilerParams(use_tc_tiling_on_sc=False),
)
def _degree_pass(dst3_hbm, zeros_hbm, out_hbm,
                 didx_all, ones_buf, acc, ssem0, ssem1):
    c = lax.axis_index("c")
    s = lax.axis_index("s")
    w = c * NS + s

    pltpu.sync_copy(zeros_hbm.at[pl.ds(s * RPT, RPT)],
                    acc.at[pl.ds(s * RPT, RPT)])
    pltpu.sync_copy(dst3_hbm.at[w], didx_all)

    def fill(j, carry):
        ones_buf[j, :] = jnp.ones((16,), jnp.float32)
        return carry
    lax.fori_loop(0, C, fill, 0)
    plsc.subcore_barrier()

    ssems = (ssem0, ssem1)

    def fire_scatter(i, p):
        pltpu.async_copy(ones_buf, acc.at[didx_all.at[i]], ssems[p], add=True)

    def drain_scatter(p):
        pltpu.make_async_copy(zeros_hbm.at[pl.ds(0, C)], ones_buf,
                              ssems[p]).wait()

    fire_scatter(0, 0)

    def body(k, carry):
        a = 2 * k + 1
        b = a + 1

        @pl.when(k > 0)
        def _():
            drain_scatter(1)
        fire_scatter(a, 1)
        drain_scatter(0)
        fire_scatter(b, 0)
        return carry

    lax.fori_loop(0, (NCH - 1) // 2, body, 0)
    drain_scatter(0)
    drain_scatter(1)
    plsc.subcore_barrier()

    pltpu.sync_copy(acc.at[pl.ds(s * RPT, RPT)],
                    out_hbm.at[c, pl.ds(s * RPT, RPT)])


F32 = jnp.float32


def _tc_call(fn, out_shapes, *args):
    def spec(a):
        return pl.BlockSpec(a.shape, lambda: tuple(0 for _ in a.shape))
    outs = out_shapes if isinstance(out_shapes, list) else [out_shapes]
    return pl.pallas_call(
        fn,
        grid=(),
        in_specs=[spec(a) for a in args],
        out_specs=(
            [pl.BlockSpec(o.shape, lambda: tuple(0 for _ in o.shape)) for o in outs]
            if isinstance(out_shapes, list)
            else pl.BlockSpec(out_shapes.shape, lambda: tuple(0 for _ in out_shapes.shape))
        ),
        out_shape=out_shapes,
    )(*args)


def _stage1(degp_ref, x_ref, w1_ref, b1_ref, disb_ref, p2_ref, u1_ref, base1_ref):
    deg = degp_ref[0, 0:N, 0:1] + degp_ref[1, 0:N, 0:1]
    dis = jnp.where(deg > 0, lax.rsqrt(jnp.where(deg > 0, deg, 1.0)), 0.0)
    disb = jnp.broadcast_to(dis, (N, HID))
    disb_ref[...] = disb
    x = x_ref[...]
    w1 = w1_ref[...]
    p2_ref[...] = disb * jnp.dot(x, w1[2], preferred_element_type=F32)
    u1_ref[...] = jnp.dot(x, w1[1], preferred_element_type=F32)
    base1_ref[...] = (
        jnp.dot(x, w1[0] - w1[2], preferred_element_type=F32) + b1_ref[0:1, :]
    )


def _stage2(u1_ref, q2p_ref, disb_ref, p1_ref):
    disb = disb_ref[...]
    q2 = q2p_ref[0, 0:N] + q2p_ref[1, 0:N]
    p1_ref[...] = disb * u1_ref[...] - 2.0 * disb * disb * q2


def _stage3(base1_ref, sp_ref, disb_ref, h_ref, ph_ref):
    disb = disb_ref[...]
    s = sp_ref[0, 0:N] + sp_ref[1, 0:N]
    h = jnp.maximum(base1_ref[...] - disb * s, 0.0)
    h_ref[...] = h
    ph_ref[...] = disb * h


def _stage4(qhp_ref, disb_ref, g1_ref, pg_ref):
    disb = disb_ref[...]
    qh = qhp_ref[0, 0:N] + qhp_ref[1, 0:N]
    g1_ref[...] = -disb * qh
    pg_ref[...] = disb * disb * qh


def _stage5(rp_ref, disb_ref, h_ref, g1_ref, w2_ref, b2_ref, o_ref):
    disb = disb_ref[...]
    g2 = disb * (rp_ref[0, 0:N] + rp_ref[1, 0:N])
    w2 = w2_ref[...]
    z = (
        jnp.dot(h_ref[...], w2[0] - w2[2], preferred_element_type=F32)
        + jnp.dot(g1_ref[...], w2[1], preferred_element_type=F32)
        + 2.0 * jnp.dot(g2, w2[2], preferred_element_type=F32)
        + b2_ref[0:1, :]
    )
    m = jnp.max(z, axis=1, keepdims=True)
    ez = jnp.exp(z - m)
    lse = jnp.log(jnp.sum(ez, axis=1, keepdims=True))
    o_ref[...] = z - m - lse


def kernel(x, edge_index, W1, b1, W2, b2):
    row3 = edge_index[0].reshape(NW, NCH, C)
    col3 = edge_index[1].reshape(NW, NCH, C)
    zeros = jnp.zeros((NP, HID), F32)
    b1r = jnp.broadcast_to(b1[None, :], (8, HID))
    b2r = jnp.broadcast_to(b2[None, :], (8, D_OUT))

    degp = _degree_pass(row3, zeros)

    disb, p2, u1, base1 = _tc_call(
        _stage1, [jax.ShapeDtypeStruct((N, HID), F32)] * 4,
        degp, x, W1, b1r)

    q2p = _scatter_pass(p2, row3, col3, zeros)

    p1 = _tc_call(_stage2, jax.ShapeDtypeStruct((N, HID), F32), u1, q2p, disb)

    sp = _scatter_pass(p1, row3, col3, zeros)

    h, ph = _tc_call(
        _stage3, [jax.ShapeDtypeStruct((N, HID), F32)] * 2,
        base1, sp, disb)

    qhp = _scatter_pass(ph, row3, col3, zeros)

    g1, pg = _tc_call(
        _stage4, [jax.ShapeDtypeStruct((N, HID), F32)] * 2,
        qhp, disb)

    rp = _scatter_pass(pg, row3, col3, zeros)

    out = _tc_call(
        _stage5, jax.ShapeDtypeStruct((N, D_OUT), F32),
        rp, disb, h, g1, W2, b2r)

    return out
